# Initial kernel scaffold; baseline (speedup 1.0000x reference)
import functools

import jax
import jax.numpy as jnp
from jax import lax
from jax.experimental import pallas as pl
from jax.experimental.pallas import tpu as pltpu
from jax.experimental.pallas import tpu_sc as plsc

Nc = 4096
Nt = 4096
NNZ = 167772
D = 1024

NQ = 4
DQ = D // NQ
N_SC = 2
N_TILES = 16
G = 128
CHUNKS = -(-NNZ // (N_TILES * G))
CH = CHUNKS * G
NNZ_PAD = N_TILES * CH
ROWS_PER_TILE = Nc // N_TILES


def _sc_body(mat_ref, row_ref, colq_ref, zeros_ref, out_ref,
             row_v, col_v, vals_v, acc, sem):
    c = lax.axis_index("c")
    s = lax.axis_index("s")

    pltpu.sync_copy(row_ref.at[s], row_v)

    for p in range(2):
        q = c * 2 + p
        pltpu.sync_copy(colq_ref.at[q, s], col_v)
        pltpu.sync_copy(zeros_ref, acc.at[pl.ds(s * ROWS_PER_TILE, ROWS_PER_TILE)])
        plsc.subcore_barrier()

        def step(j, carry):
            pltpu.async_copy(mat_ref.at[col_v.at[j]], vals_v, sem).wait()
            pltpu.sync_copy(vals_v, acc.at[row_v.at[j]], add=True)
            return carry

        lax.fori_loop(0, CHUNKS, step, 0)
        plsc.subcore_barrier()
        pltpu.sync_copy(
            acc.at[pl.ds(s * ROWS_PER_TILE, ROWS_PER_TILE)],
            out_ref.at[q, pl.ds(s * ROWS_PER_TILE, ROWS_PER_TILE)],
        )


_sc_call = functools.partial(
    pl.kernel,
    out_type=jax.ShapeDtypeStruct((NQ, Nc, DQ), jnp.float32),
    mesh=plsc.VectorSubcoreMesh(core_axis_name="c", subcore_axis_name="s"),
    scratch_types=[
        pltpu.VMEM((CHUNKS, G), jnp.int32),
        pltpu.VMEM((CHUNKS, G), jnp.int32),
        pltpu.VMEM((G, DQ), jnp.float32),
        pltpu.VMEM_SHARED((Nc + 1, DQ), jnp.float32),
        pltpu.SemaphoreType.DMA,
    ],
)(_sc_body)


def kernel(mat, row, col):
    pad = NNZ_PAD - NNZ
    row_p = jnp.concatenate([row, jnp.full((pad,), Nc, jnp.int32)])
    row_p = row_p.reshape(N_TILES, CHUNKS, G)
    col_p = jnp.concatenate([col, jnp.zeros((pad,), jnp.int32)])
    colq = col_p[None, :] * NQ + jnp.arange(NQ, dtype=jnp.int32)[:, None]
    colq = colq.reshape(NQ, N_TILES, CHUNKS, G)
    mat_r = mat.reshape(Nt * NQ, DQ)
    zeros = jnp.zeros((ROWS_PER_TILE, DQ), jnp.float32)
    out4 = _sc_call(mat_r, row_p, colq, zeros)
    return out4.transpose(1, 0, 2).reshape(Nc, D)


# speedup vs baseline: 3.8205x; 3.8205x over previous
import functools

import jax
import jax.numpy as jnp
from jax import lax
from jax.experimental import pallas as pl
from jax.experimental.pallas import tpu as pltpu
from jax.experimental.pallas import tpu_sc as plsc

Nc = 4096
Nt = 4096
NNZ = 167772
D = 1024

NG = 8
DG = D // NG
N_TILES = 16
G = 128
CHUNKS = -(-NNZ // (N_TILES * G))
NNZ_PAD = N_TILES * CHUNKS * G
ACC_ROWS = Nc + 8
RPT = Nc // N_TILES
N_PASS = 4


def _sc_body(mat_ref, ridx_ref, colg_ref, zeros_ref, out_ref,
             ridx_v, cidx_v, vals_v, acc, gsem, ssem):
    c = lax.axis_index("c")
    s = lax.axis_index("s")

    pltpu.sync_copy(ridx_ref.at[s], ridx_v)

    for p in range(N_PASS):
        g = c * N_PASS + p
        pltpu.sync_copy(colg_ref.at[g, s], cidx_v)
        pltpu.sync_copy(zeros_ref, acc.at[pl.ds(s * RPT, RPT)])
        plsc.subcore_barrier()

        def step(j, carry):
            pltpu.async_copy(mat_ref.at[cidx_v.at[j]], vals_v, gsem).wait()
            pltpu.async_copy(vals_v, acc.at[ridx_v.at[j]], ssem,
                             add=True).wait()
            return carry

        lax.fori_loop(0, CHUNKS, step, 0)
        plsc.subcore_barrier()
        pltpu.sync_copy(acc.at[pl.ds(s * RPT, RPT)],
                        out_ref.at[g, pl.ds(s * RPT, RPT)])


_sc_call = functools.partial(
    pl.kernel,
    out_type=jax.ShapeDtypeStruct((NG, Nc, DG), jnp.float32),
    mesh=plsc.VectorSubcoreMesh(core_axis_name="c", subcore_axis_name="s"),
    scratch_types=[
        pltpu.VMEM((CHUNKS, G), jnp.int32),
        pltpu.VMEM((CHUNKS, G), jnp.int32),
        pltpu.VMEM((G, DG), jnp.float32),
        pltpu.VMEM_SHARED((ACC_ROWS, DG), jnp.float32),
        pltpu.SemaphoreType.DMA,
        pltpu.SemaphoreType.DMA,
    ],
)(_sc_body)


def kernel(mat, row, col):
    pad = NNZ_PAD - NNZ
    row_p = jnp.concatenate([row, jnp.full((pad,), Nc, jnp.int32)])
    col_p = jnp.concatenate([col, jnp.zeros((pad,), jnp.int32)])
    ridx = row_p.reshape(N_TILES, CHUNKS, G)
    gs = jnp.arange(NG, dtype=jnp.int32)[:, None]
    colg = (col_p[None, :] * NG + gs).reshape(NG, N_TILES, CHUNKS, G)
    mat_r = mat.reshape(Nt * NG, DG)
    zeros = jnp.zeros((RPT, DG), jnp.float32)
    out8 = _sc_call(mat_r, ridx, colg, zeros)
    return out8.transpose(1, 0, 2).reshape(Nc, D)
